# baseline (speedup 1.0000x reference)
import functools

import jax
import jax.numpy as jnp
from jax import lax
from jax.experimental import pallas as pl
from jax.experimental.pallas import tpu as pltpu
from jax.experimental.pallas import tpu_sc as plsc

N = 10000
E = 320000
D = 16
NP = 10240
NTILES = 32
CH = 128
NCH = 80
EPT = NCH * CH
EPAD = NTILES * EPT
DUMMY = N + 16
RPT = NP // 16
RPN = N // 16

NBUF = 4
CG = 4
NG = NCH // CG
NT = NG // NBUF

_f32 = jnp.float32


def _stage_edges_and_zero(srcs_hbm, dsts_hbm, zeros_hbm, src_v, dst_v,
                          agg_sh, tid, s):
    pltpu.sync_copy(srcs_hbm.at[tid], src_v)
    pltpu.sync_copy(dsts_hbm.at[tid], dst_v)
    pltpu.sync_copy(zeros_hbm.at[pl.ds(s * RPT, RPT)],
                    agg_sh.at[pl.ds(s * RPT, RPT)])


def _edge_pipeline(src_v, dst_v, msg_v, g_sh, agg_sh, gsem, ssem):
    for b in range(NBUF):
        pltpu.async_copy(g_sh.at[src_v.at[b]], msg_v.at[b], gsem)

    def body(t, carry):
        for b in range(NBUF):
            j = t * NBUF + b
            pltpu.make_async_copy(g_sh.at[src_v.at[j]], msg_v.at[b],
                                  gsem).wait()
            pltpu.async_copy(msg_v.at[b], agg_sh.at[dst_v.at[j]], ssem,
                             add=True)

            @pl.when(t < NT - 1)
            def _():
                pltpu.make_async_copy(msg_v.at[b], agg_sh.at[dst_v.at[j]],
                                      ssem).wait()
                pltpu.async_copy(g_sh.at[src_v.at[j + NBUF]], msg_v.at[b],
                                 gsem)
        return carry

    lax.fori_loop(0, NT, body, 0)
    for b in range(NBUF):
        pltpu.make_async_copy(msg_v.at[b], agg_sh.at[dst_v.at[NG - NBUF + b]],
                              ssem).wait()


def _sc_propagate_body(g_hbm, srcs_hbm, dsts_hbm, zeros_hbm, out_hbm,
                       src_v, dst_v, msg_v, g_sh, agg_sh, gsem, ssem):
    c = lax.axis_index("c")
    s = lax.axis_index("s")
    _stage_edges_and_zero(srcs_hbm, dsts_hbm, zeros_hbm, src_v, dst_v,
                          agg_sh, s * 2 + c, s)
    pltpu.sync_copy(g_hbm.at[pl.ds(s * RPT, RPT)],
                    g_sh.at[pl.ds(s * RPT, RPT)])
    plsc.subcore_barrier()
    _edge_pipeline(src_v, dst_v, msg_v, g_sh, agg_sh, gsem, ssem)
    plsc.subcore_barrier()
    pltpu.sync_copy(agg_sh.at[pl.ds(s * RPT, RPT)],
                    out_hbm.at[c].at[pl.ds(s * RPT, RPT)])


_sc_propagate = functools.partial(
    pl.kernel,
    out_type=jax.ShapeDtypeStruct((2, NP, D), _f32),
    mesh=plsc.VectorSubcoreMesh(core_axis_name="c", subcore_axis_name="s"),
    scratch_types=[
        pltpu.VMEM((NG, CG * CH), jnp.int32),
        pltpu.VMEM((NG, CG * CH), jnp.int32),
        pltpu.VMEM((NBUF, CG * CH, D), _f32),
        pltpu.VMEM_SHARED((NP, D), _f32),
        pltpu.VMEM_SHARED((NP, D), _f32),
        pltpu.SemaphoreType.DMA,
        pltpu.SemaphoreType.DMA,
    ],
    compiler_params=pltpu.CompilerParams(use_tc_tiling_on_sc=False, needs_layout_passes=False),
)(_sc_propagate_body)


def _sc_layer_body(s_hbm, g_hbm, dinv_hbm, wspl_hbm, b_hbm,
                   srcs_hbm, dsts_hbm, zeros_hbm, sout_hbm, gout_hbm,
                   src_v, dst_v, msg_v, s0_v, s1_v, gp_v, di_v, gbuf_v,
                   wsplat_v, b_v, g_sh, agg_sh, gsem, ssem):
    c = lax.axis_index("c")
    s = lax.axis_index("s")
    base = s * RPT
    _stage_edges_and_zero(srcs_hbm, dsts_hbm, zeros_hbm, src_v, dst_v,
                          agg_sh, s * 2 + c, s)
    cps = [(s_hbm.at[0].at[pl.ds(base, RPT)], s0_v),
           (s_hbm.at[1].at[pl.ds(base, RPT)], s1_v),
           (g_hbm.at[pl.ds(base, RPT)], gp_v),
           (dinv_hbm.at[pl.ds(base, RPT)], di_v),
           (wspl_hbm, wsplat_v),
           (b_hbm, b_v)]
    for sref, dref in cps:
        pltpu.async_copy(sref, dref, gsem)
    for sref, dref in cps:
        pltpu.make_async_copy(sref, dref, gsem).wait()

    bvec = b_v[...]

    def elw(r, carry):
        dv = di_v[r]
        s0_v[r] = jnp.maximum(
            (s0_v[r] + s1_v[r] + gp_v[r]) * dv + bvec, 0.0) * dv
        return carry

    lax.fori_loop(0, RPT, elw, 0)

    iota = lax.iota(jnp.int32, D)
    colc = [jnp.full((D,), j, jnp.int32) for j in range(D)]

    for kc in range(D // 2):
        k0, k1 = 2 * kc, 2 * kc + 1
        w0 = [wsplat_v[k0 * D + j] for j in range(D)]
        w1 = [wsplat_v[k1 * D + j] for j in range(D)]

        def mm(gi, carry, k0=k0, k1=k1, w0=w0, w1=w1, first=(kc == 0)):
            rowv = gi * D + iota
            col0 = plsc.load_gather(s0_v, [rowv, colc[k0]])
            col1 = plsc.load_gather(s0_v, [rowv, colc[k1]])
            for j in range(D):
                t = col0 * w0[j] + col1 * w1[j]
                if first:
                    plsc.store_scatter(gbuf_v, [rowv, colc[j]], t)
                else:
                    plsc.addupdate_scatter(gbuf_v, [rowv, colc[j]], t)
            return carry

        lax.fori_loop(0, RPT // D, mm, 0)

    pltpu.sync_copy(gbuf_v, g_sh.at[pl.ds(base, RPT)])

    @pl.when(c == 0)
    def _():
        pltpu.sync_copy(gbuf_v, gout_hbm.at[pl.ds(base, RPT)])

    plsc.subcore_barrier()
    _edge_pipeline(src_v, dst_v, msg_v, g_sh, agg_sh, gsem, ssem)
    plsc.subcore_barrier()
    pltpu.sync_copy(agg_sh.at[pl.ds(base, RPT)],
                    sout_hbm.at[c].at[pl.ds(base, RPT)])


_sc_layer = functools.partial(
    pl.kernel,
    out_type=[jax.ShapeDtypeStruct((2, NP, D), _f32),
              jax.ShapeDtypeStruct((NP, D), _f32)],
    mesh=plsc.VectorSubcoreMesh(core_axis_name="c", subcore_axis_name="s"),
    scratch_types=[
        pltpu.VMEM((NG, CG * CH), jnp.int32),
        pltpu.VMEM((NG, CG * CH), jnp.int32),
        pltpu.VMEM((NBUF, CG * CH, D), _f32),
        pltpu.VMEM((RPT, D), _f32),
        pltpu.VMEM((RPT, D), _f32),
        pltpu.VMEM((RPT, D), _f32),
        pltpu.VMEM((RPT, D), _f32),
        pltpu.VMEM((RPT, D), _f32),
        pltpu.VMEM((D * D, D), _f32),
        pltpu.VMEM((D,), _f32),
        pltpu.VMEM_SHARED((NP, D), _f32),
        pltpu.VMEM_SHARED((NP, D), _f32),
        pltpu.SemaphoreType.DMA,
        pltpu.SemaphoreType.DMA,
    ],
    compiler_params=pltpu.CompilerParams(use_tc_tiling_on_sc=False, needs_layout_passes=False),
)(_sc_layer_body)


def _sc_final_body(s_hbm, g_hbm, dinv_hbm, b_hbm, out_hbm,
                   s0_v, s1_v, gp_v, di_v, b_v):
    c = lax.axis_index("c")
    s = lax.axis_index("s")

    @pl.when(c == 0)
    def _():
        base = s * RPN
        pltpu.sync_copy(s_hbm.at[0].at[pl.ds(base, RPN)], s0_v)
        pltpu.sync_copy(s_hbm.at[1].at[pl.ds(base, RPN)], s1_v)
        pltpu.sync_copy(g_hbm.at[pl.ds(base, RPN)], gp_v)
        pltpu.sync_copy(dinv_hbm.at[pl.ds(base, RPN)], di_v)
        pltpu.sync_copy(b_hbm, b_v)
        bvec = b_v[...]

        def elw(r, carry):
            s0_v[r] = (s0_v[r] + s1_v[r] + gp_v[r]) * di_v[r] + bvec
            return carry

        lax.fori_loop(0, RPN, elw, 0)
        pltpu.sync_copy(s0_v, out_hbm.at[pl.ds(base, RPN)])


_sc_final = functools.partial(
    pl.kernel,
    out_type=jax.ShapeDtypeStruct((N, D), _f32),
    mesh=plsc.VectorSubcoreMesh(core_axis_name="c", subcore_axis_name="s"),
    scratch_types=[
        pltpu.VMEM((RPN, D), _f32),
        pltpu.VMEM((RPN, D), _f32),
        pltpu.VMEM((RPN, D), _f32),
        pltpu.VMEM((RPN, D), _f32),
        pltpu.VMEM((D,), _f32),
    ],
    compiler_params=pltpu.CompilerParams(use_tc_tiling_on_sc=False, needs_layout_passes=False),
)(_sc_final_body)


def _tc_first_body(a_ref, x_ref, w_ref, dinv_ref, g_ref):
    dinv = lax.rsqrt(a_ref[0] + a_ref[1] + 1.0)
    dinv_ref[...] = dinv
    g_ref[...] = jnp.dot(x_ref[...], w_ref[...],
                         preferred_element_type=jnp.float32) * dinv


_tc_first = pl.pallas_call(
    _tc_first_body,
    out_shape=[jax.ShapeDtypeStruct((NP, D), _f32),
               jax.ShapeDtypeStruct((NP, D), _f32)])


def kernel(x, edge_index, W0, b0, W1, b1, W2, b2, W3, b3, W4, b4, W5, b5,
           W6, b6, W7, b7):
    Ws = [W0, W1, W2, W3, W4, W5, W6, W7]
    bs = [b0, b1, b2, b3, b4, b5, b6, b7]

    src = edge_index[0]
    dst = edge_index[1]
    pad = EPAD - E
    srcs = jnp.concatenate(
        [src, jnp.full((pad,), DUMMY, jnp.int32)]).reshape(NTILES, NG, CG * CH)
    dsts = jnp.concatenate(
        [dst, jnp.full((pad,), DUMMY, jnp.int32)]).reshape(NTILES, NG, CG * CH)
    zeros = jnp.zeros((NP, D), _f32)
    ones = jnp.ones((NP, D), _f32)
    x_p = jnp.pad(x, ((0, NP - N), (0, 0)))

    aggones = _sc_propagate(ones, srcs, dsts, zeros)

    dinv, g = _tc_first(aggones, x_p, W0)

    s = _sc_propagate(g, srcs, dsts, zeros)
    for i in range(1, 8):
        wspl = jnp.broadcast_to(Ws[i].reshape(D * D, 1), (D * D, D))
        s, g = _sc_layer(s, g, dinv, wspl, bs[i - 1], srcs, dsts, zeros)

    return _sc_final(s, g, dinv, bs[7])


# speedup vs baseline: 1.3483x; 1.3483x over previous
import functools

import jax
import jax.numpy as jnp
from jax import lax
from jax.experimental import pallas as pl
from jax.experimental.pallas import tpu as pltpu
from jax.experimental.pallas import tpu_sc as plsc

N = 10000
E = 320000
D = 16
NP = 10240
NTILES = 32
CH = 128
NCH = 80
EPT = NCH * CH
EPAD = NTILES * EPT
DUMMY = N + 16
RPT = NP // 16
PK = NP // 8


NBUF = 4
CG = 4
NG = NCH // CG
NT = NG // NBUF


def _sc_propagate_body(g_hbm, srcs_hbm, dsts_hbm, zeros_hbm, out_hbm,
                       src_v, dst_v, msg_v, g_sh, agg_sh, gsem, ssem):
    c = lax.axis_index("c")
    s = lax.axis_index("s")
    tid = s * 2 + c
    pltpu.sync_copy(srcs_hbm.at[tid], src_v)
    pltpu.sync_copy(dsts_hbm.at[tid], dst_v)
    pltpu.sync_copy(g_hbm.at[pl.ds(s * RPT, RPT)],
                    g_sh.at[pl.ds(s * RPT, RPT)])
    pltpu.sync_copy(zeros_hbm.at[pl.ds(s * RPT, RPT)],
                    agg_sh.at[pl.ds(s * RPT, RPT)])
    plsc.subcore_barrier()
    for b in range(NBUF):
        pltpu.async_copy(g_sh.at[src_v.at[b]], msg_v.at[b], gsem)

    def body(t, carry):
        for b in range(NBUF):
            j = t * NBUF + b
            pltpu.make_async_copy(g_sh.at[src_v.at[j]], msg_v.at[b],
                                  gsem).wait()
            pltpu.async_copy(msg_v.at[b], agg_sh.at[dst_v.at[j]], ssem,
                             add=True)

            @pl.when(t < NT - 1)
            def _():
                pltpu.make_async_copy(msg_v.at[b], agg_sh.at[dst_v.at[j]],
                                      ssem).wait()
                pltpu.async_copy(g_sh.at[src_v.at[j + NBUF]], msg_v.at[b],
                                 gsem)
        return carry

    lax.fori_loop(0, NT, body, 0)
    for b in range(NBUF):
        pltpu.make_async_copy(msg_v.at[b], agg_sh.at[dst_v.at[NG - NBUF + b]],
                              ssem).wait()
    plsc.subcore_barrier()
    pltpu.sync_copy(agg_sh.at[pl.ds(s * RPT, RPT)],
                    out_hbm.at[c].at[pl.ds(s * RPT, RPT)])


_sc_propagate = functools.partial(
    pl.kernel,
    out_type=jax.ShapeDtypeStruct((2, NP, D), jnp.float32),
    mesh=plsc.VectorSubcoreMesh(core_axis_name="c", subcore_axis_name="s"),
    scratch_types=[
        pltpu.VMEM((NG, CG * CH), jnp.int32),
        pltpu.VMEM((NG, CG * CH), jnp.int32),
        pltpu.VMEM((NBUF, CG * CH, D), jnp.float32),
        pltpu.VMEM_SHARED((NP, D), jnp.float32),
        pltpu.VMEM_SHARED((NP, D), jnp.float32),
        pltpu.SemaphoreType.DMA,
        pltpu.SemaphoreType.DMA,
    ],
    compiler_params=pltpu.CompilerParams(use_tc_tiling_on_sc=False),
)(_sc_propagate_body)


def _propagate(g, srcs, dsts, zeros):
    return _sc_propagate(g, srcs, dsts, zeros)


def _tc_first_body(a0_ref, a1_ref, x_ref, w_ref, dinv_ref, g_ref):
    dinv = lax.rsqrt(a0_ref[...] + a1_ref[...] + 1.0)
    dinv_ref[...] = dinv
    g_ref[...] = jnp.dot(x_ref[...], w_ref[...],
                         preferred_element_type=jnp.float32) * dinv


def _tc_mid_body(s0_ref, s1_ref, g_ref, dinv_ref, b_ref, w_ref, out_ref):
    dinv = dinv_ref[...]
    h = jnp.maximum((s0_ref[...] + s1_ref[...] + g_ref[...]) * dinv
                    + b_ref[...], 0.0)
    out_ref[...] = jnp.dot(h, w_ref[...],
                           preferred_element_type=jnp.float32) * dinv


def _tc_final_body(s0_ref, s1_ref, g_ref, dinv_ref, b_ref, out_ref):
    out_ref[...] = ((s0_ref[...] + s1_ref[...] + g_ref[...]) * dinv_ref[...]
                    + b_ref[...])


_f32 = jnp.float32
_tc_first = pl.pallas_call(
    _tc_first_body,
    out_shape=[jax.ShapeDtypeStruct((PK, 128), _f32),
               jax.ShapeDtypeStruct((PK, 128), _f32)])
_tc_mid = pl.pallas_call(
    _tc_mid_body, out_shape=jax.ShapeDtypeStruct((PK, 128), _f32))
_tc_final = pl.pallas_call(
    _tc_final_body, out_shape=jax.ShapeDtypeStruct((PK, 128), _f32))


def _blockdiag(w):
    k = w.shape[0]
    return jnp.einsum("pq,kj->pkqj", jnp.eye(8, dtype=w.dtype),
                      w).reshape(8 * k, 128)


def kernel(x, edge_index, W0, b0, W1, b1, W2, b2, W3, b3, W4, b4, W5, b5,
           W6, b6, W7, b7):
    Ws = [W0, W1, W2, W3, W4, W5, W6, W7]
    bs = [b0, b1, b2, b3, b4, b5, b6, b7]

    src = edge_index[0]
    dst = edge_index[1]
    pad = EPAD - E
    srcs = jnp.concatenate(
        [src, jnp.full((pad,), DUMMY, jnp.int32)]).reshape(NTILES, NG, CG * CH)
    dsts = jnp.concatenate(
        [dst, jnp.full((pad,), DUMMY, jnp.int32)]).reshape(NTILES, NG, CG * CH)
    zeros = jnp.zeros((NP, D), _f32)
    ones = jnp.ones((NP, D), _f32)
    x_pp = jnp.pad(x, ((0, NP - N), (0, 0))).reshape(PK, 1024)

    w0big = _blockdiag(W0)
    wbigs = [_blockdiag(w) for w in Ws[1:]]
    btiles = [jnp.tile(b, 8).reshape(1, 128) for b in bs]

    aggones = _propagate(ones, srcs, dsts, zeros).reshape(2, PK, 128)

    dinv_p, g_p = _tc_first(aggones[0], aggones[1], x_pp, w0big)

    for i in range(8):
        sp = _propagate(g_p.reshape(NP, D), srcs, dsts, zeros)
        sp = sp.reshape(2, PK, 128)
        if i < 7:
            g_p = _tc_mid(sp[0], sp[1], g_p, dinv_p, btiles[i], wbigs[i])
        else:
            out_p = _tc_final(sp[0], sp[1], g_p, dinv_p, btiles[i])

    return out_p.reshape(NP, D)[:N]


# speedup vs baseline: 1.3673x; 1.0141x over previous
import functools

import jax
import jax.numpy as jnp
from jax import lax
from jax.experimental import pallas as pl
from jax.experimental.pallas import tpu as pltpu
from jax.experimental.pallas import tpu_sc as plsc

N = 10000
E = 320000
D = 16
NP = 10240
NTILES = 32
CH = 128
NCH = 80
EPT = NCH * CH
EPAD = NTILES * EPT
DUMMY = N + 16
RPT = NP // 16
RPN = N // 16
PK = NP // 8


NBUF = 4
CG = 4
NG = NCH // CG
NT = NG // NBUF


def _sc_propagate_body(g_hbm, srcs_hbm, dsts_hbm, zeros_hbm, out_hbm,
                       src_v, dst_v, msg_v, g_sh, agg_sh, gsem, ssem):
    c = lax.axis_index("c")
    s = lax.axis_index("s")
    tid = s * 2 + c
    pltpu.sync_copy(srcs_hbm.at[tid], src_v)
    pltpu.sync_copy(dsts_hbm.at[tid], dst_v)
    pltpu.sync_copy(g_hbm.at[pl.ds(s * RPT, RPT)],
                    g_sh.at[pl.ds(s * RPT, RPT)])
    pltpu.sync_copy(zeros_hbm.at[pl.ds(s * RPT, RPT)],
                    agg_sh.at[pl.ds(s * RPT, RPT)])
    plsc.subcore_barrier()
    for b in range(NBUF):
        pltpu.async_copy(g_sh.at[src_v.at[b]], msg_v.at[b], gsem)

    def body(t, carry):
        for b in range(NBUF):
            j = t * NBUF + b
            pltpu.make_async_copy(g_sh.at[src_v.at[j]], msg_v.at[b],
                                  gsem).wait()
            pltpu.async_copy(msg_v.at[b], agg_sh.at[dst_v.at[j]], ssem,
                             add=True)

            @pl.when(t < NT - 1)
            def _():
                pltpu.make_async_copy(msg_v.at[b], agg_sh.at[dst_v.at[j]],
                                      ssem).wait()
                pltpu.async_copy(g_sh.at[src_v.at[j + NBUF]], msg_v.at[b],
                                 gsem)
        return carry

    lax.fori_loop(0, NT, body, 0)
    for b in range(NBUF):
        pltpu.make_async_copy(msg_v.at[b], agg_sh.at[dst_v.at[NG - NBUF + b]],
                              ssem).wait()
    plsc.subcore_barrier()
    pltpu.sync_copy(agg_sh.at[pl.ds(s * RPT, RPT)],
                    out_hbm.at[c].at[pl.ds(s * RPT, RPT)])


_sc_propagate = functools.partial(
    pl.kernel,
    out_type=jax.ShapeDtypeStruct((2, NP, D), jnp.float32),
    mesh=plsc.VectorSubcoreMesh(core_axis_name="c", subcore_axis_name="s"),
    scratch_types=[
        pltpu.VMEM((NG, CG * CH), jnp.int32),
        pltpu.VMEM((NG, CG * CH), jnp.int32),
        pltpu.VMEM((NBUF, CG * CH, D), jnp.float32),
        pltpu.VMEM_SHARED((NP, D), jnp.float32),
        pltpu.VMEM_SHARED((NP, D), jnp.float32),
        pltpu.SemaphoreType.DMA,
        pltpu.SemaphoreType.DMA,
    ],
    compiler_params=pltpu.CompilerParams(use_tc_tiling_on_sc=False),
)(_sc_propagate_body)


def _propagate(g, srcs, dsts, zeros):
    return _sc_propagate(g, srcs, dsts, zeros)


def _sc_final_body(s_hbm, g_hbm, dinv_hbm, b_hbm, out_hbm,
                   s0_v, s1_v, gp_v, di_v, b_v):
    c = lax.axis_index("c")
    s = lax.axis_index("s")

    @pl.when(c == 0)
    def _():
        base = s * RPN
        pltpu.sync_copy(s_hbm.at[0].at[pl.ds(base, RPN)], s0_v)
        pltpu.sync_copy(s_hbm.at[1].at[pl.ds(base, RPN)], s1_v)
        pltpu.sync_copy(g_hbm.at[pl.ds(base, RPN)], gp_v)
        pltpu.sync_copy(dinv_hbm.at[pl.ds(base, RPN)], di_v)
        pltpu.sync_copy(b_hbm, b_v)
        bvec = b_v[...]

        def elw(r, carry):
            s0_v[r] = (s0_v[r] + s1_v[r] + gp_v[r]) * di_v[r] + bvec
            return carry

        lax.fori_loop(0, RPN, elw, 0)
        pltpu.sync_copy(s0_v, out_hbm.at[pl.ds(base, RPN)])


_sc_final = functools.partial(
    pl.kernel,
    out_type=jax.ShapeDtypeStruct((N, D), jnp.float32),
    mesh=plsc.VectorSubcoreMesh(core_axis_name="c", subcore_axis_name="s"),
    scratch_types=[
        pltpu.VMEM((RPN, D), jnp.float32),
        pltpu.VMEM((RPN, D), jnp.float32),
        pltpu.VMEM((RPN, D), jnp.float32),
        pltpu.VMEM((RPN, D), jnp.float32),
        pltpu.VMEM((D,), jnp.float32),
    ],
    compiler_params=pltpu.CompilerParams(use_tc_tiling_on_sc=False,
                                         needs_layout_passes=False),
)(_sc_final_body)


def _tc_first_body(a0_ref, a1_ref, x_ref, w_ref, dinv_ref, g_ref):
    dinv = lax.rsqrt(a0_ref[...] + a1_ref[...] + 1.0)
    dinv_ref[...] = dinv
    g_ref[...] = jnp.dot(x_ref[...], w_ref[...],
                         preferred_element_type=jnp.float32) * dinv


def _tc_mid_body(s0_ref, s1_ref, g_ref, dinv_ref, b_ref, w_ref, out_ref):
    dinv = dinv_ref[...]
    h = jnp.maximum((s0_ref[...] + s1_ref[...] + g_ref[...]) * dinv
                    + b_ref[...], 0.0)
    out_ref[...] = jnp.dot(h, w_ref[...],
                           preferred_element_type=jnp.float32) * dinv


def _tc_final_body(s0_ref, s1_ref, g_ref, dinv_ref, b_ref, out_ref):
    out_ref[...] = ((s0_ref[...] + s1_ref[...] + g_ref[...]) * dinv_ref[...]
                    + b_ref[...])


_f32 = jnp.float32
_tc_first = pl.pallas_call(
    _tc_first_body,
    out_shape=[jax.ShapeDtypeStruct((PK, 128), _f32),
               jax.ShapeDtypeStruct((PK, 128), _f32)])
_tc_mid = pl.pallas_call(
    _tc_mid_body, out_shape=jax.ShapeDtypeStruct((PK, 128), _f32))
_tc_final = pl.pallas_call(
    _tc_final_body, out_shape=jax.ShapeDtypeStruct((PK, 128), _f32))


def _blockdiag(w):
    k = w.shape[0]
    return jnp.einsum("pq,kj->pkqj", jnp.eye(8, dtype=w.dtype),
                      w).reshape(8 * k, 128)


def kernel(x, edge_index, W0, b0, W1, b1, W2, b2, W3, b3, W4, b4, W5, b5,
           W6, b6, W7, b7):
    Ws = [W0, W1, W2, W3, W4, W5, W6, W7]
    bs = [b0, b1, b2, b3, b4, b5, b6, b7]

    src = edge_index[0]
    dst = edge_index[1]
    pad = EPAD - E
    srcs = jnp.concatenate(
        [src, jnp.full((pad,), DUMMY, jnp.int32)]).reshape(NTILES, NG, CG * CH)
    dsts = jnp.concatenate(
        [dst, jnp.full((pad,), DUMMY, jnp.int32)]).reshape(NTILES, NG, CG * CH)
    zeros = jnp.zeros((NP, D), _f32)
    ones = jnp.ones((NP, D), _f32)
    x_pp = jnp.pad(x, ((0, NP - N), (0, 0))).reshape(PK, 1024)

    w0big = _blockdiag(W0)
    wbigs = [_blockdiag(w) for w in Ws[1:]]
    btiles = [jnp.tile(b, 8).reshape(1, 128) for b in bs]

    aggones = _propagate(ones, srcs, dsts, zeros).reshape(2, PK, 128)

    dinv_p, g_p = _tc_first(aggones[0], aggones[1], x_pp, w0big)

    for i in range(8):
        g_flat = g_p.reshape(NP, D)
        sp_raw = _propagate(g_flat, srcs, dsts, zeros)
        if i < 7:
            sp = sp_raw.reshape(2, PK, 128)
            g_p = _tc_mid(sp[0], sp[1], g_p, dinv_p, btiles[i], wbigs[i])
        else:
            out = _sc_final(sp_raw, g_flat, dinv_p.reshape(NP, D), bs[i])

    return out


# speedup vs baseline: 1.3695x; 1.0016x over previous
import functools

import jax
import jax.numpy as jnp
from jax import lax
from jax.experimental import pallas as pl
from jax.experimental.pallas import tpu as pltpu
from jax.experimental.pallas import tpu_sc as plsc

N = 10000
E = 320000
D = 16
NP = 10240
NTILES = 32
CH = 128
NCH = 80
EPT = NCH * CH
EPAD = NTILES * EPT
DUMMY = N + 16
RPT = NP // 16
RPN = N // 16
PK = NP // 8


NBUF = 4
CG = 4
NG = NCH // CG
NT = NG // NBUF


def _sc_propagate_body(g_hbm, srcs_hbm, dsts_hbm, zeros_hbm, out_hbm,
                       src_v, dst_v, msg_v, g_sh, agg_sh, gsem, ssem):
    c = lax.axis_index("c")
    s = lax.axis_index("s")
    tid = s * 2 + c
    pltpu.sync_copy(srcs_hbm.at[tid], src_v)
    pltpu.sync_copy(dsts_hbm.at[tid], dst_v)
    pltpu.sync_copy(g_hbm.at[pl.ds(s * RPT, RPT)],
                    g_sh.at[pl.ds(s * RPT, RPT)])
    pltpu.sync_copy(zeros_hbm.at[pl.ds(s * RPT, RPT)],
                    agg_sh.at[pl.ds(s * RPT, RPT)])
    plsc.subcore_barrier()
    for b in range(NBUF):
        pltpu.async_copy(g_sh.at[src_v.at[b]], msg_v.at[b], gsem)

    def body(t, carry):
        for b in range(NBUF):
            j = t * NBUF + b
            pltpu.make_async_copy(g_sh.at[src_v.at[j]], msg_v.at[b],
                                  gsem).wait()
            pltpu.async_copy(msg_v.at[b], agg_sh.at[dst_v.at[j]], ssem,
                             add=True)

            @pl.when(t < NT - 1)
            def _():
                pltpu.make_async_copy(msg_v.at[b], agg_sh.at[dst_v.at[j]],
                                      ssem).wait()
                pltpu.async_copy(g_sh.at[src_v.at[j + NBUF]], msg_v.at[b],
                                 gsem)
        return carry

    lax.fori_loop(0, NT, body, 0)
    for b in range(NBUF):
        pltpu.make_async_copy(msg_v.at[b], agg_sh.at[dst_v.at[NG - NBUF + b]],
                              ssem).wait()
    plsc.subcore_barrier()
    pltpu.sync_copy(agg_sh.at[pl.ds(s * RPT, RPT)],
                    out_hbm.at[c].at[pl.ds(s * RPT, RPT)])


_sc_propagate = functools.partial(
    pl.kernel,
    out_type=jax.ShapeDtypeStruct((2, NP, D), jnp.float32),
    mesh=plsc.VectorSubcoreMesh(core_axis_name="c", subcore_axis_name="s"),
    scratch_types=[
        pltpu.VMEM((NG, CG * CH), jnp.int32),
        pltpu.VMEM((NG, CG * CH), jnp.int32),
        pltpu.VMEM((NBUF, CG * CH, D), jnp.float32),
        pltpu.VMEM_SHARED((NP, D), jnp.float32),
        pltpu.VMEM_SHARED((NP, D), jnp.float32),
        pltpu.SemaphoreType.DMA,
        pltpu.SemaphoreType.DMA,
    ],
    compiler_params=pltpu.CompilerParams(use_tc_tiling_on_sc=False),
)(_sc_propagate_body)


def _propagate(g, srcs, dsts, zeros):
    return _sc_propagate(g, srcs, dsts, zeros)


def _sc_final_body(s_hbm, g_hbm, dinv_hbm, b_hbm, out_hbm,
                   s0_v, s1_v, gp_v, di_v, b_v):
    c = lax.axis_index("c")
    s = lax.axis_index("s")

    @pl.when(c == 0)
    def _():
        base = s * RPN
        pltpu.sync_copy(s_hbm.at[0].at[pl.ds(base, RPN)], s0_v)
        pltpu.sync_copy(s_hbm.at[1].at[pl.ds(base, RPN)], s1_v)
        pltpu.sync_copy(g_hbm.at[pl.ds(base, RPN)], gp_v)
        pltpu.sync_copy(dinv_hbm.at[pl.ds(base, RPN)], di_v)
        pltpu.sync_copy(b_hbm, b_v)
        bvec = b_v[...]

        def elw(r, carry):
            s0_v[r] = (s0_v[r] + s1_v[r] + gp_v[r]) * di_v[r] + bvec
            return carry

        lax.fori_loop(0, RPN, elw, 0)
        pltpu.sync_copy(s0_v, out_hbm.at[pl.ds(base, RPN)])


_sc_final = functools.partial(
    pl.kernel,
    out_type=jax.ShapeDtypeStruct((N, D), jnp.float32),
    mesh=plsc.VectorSubcoreMesh(core_axis_name="c", subcore_axis_name="s"),
    scratch_types=[
        pltpu.VMEM((RPN, D), jnp.float32),
        pltpu.VMEM((RPN, D), jnp.float32),
        pltpu.VMEM((RPN, D), jnp.float32),
        pltpu.VMEM((RPN, D), jnp.float32),
        pltpu.VMEM((D,), jnp.float32),
    ],
    compiler_params=pltpu.CompilerParams(use_tc_tiling_on_sc=False,
                                         needs_layout_passes=False),
)(_sc_final_body)


def _tc_first_body(a0_ref, a1_ref, x_ref, w_ref, dinv_ref, g_ref):
    dinv = lax.rsqrt(a0_ref[...] + a1_ref[...] + 1.0)
    dinv_ref[...] = dinv
    g_ref[...] = jnp.dot(x_ref[...], w_ref[...],
                         preferred_element_type=jnp.float32) * dinv


def _tc_mid_body(s0_ref, s1_ref, g_ref, dinv_ref, b_ref, w_ref, out_ref):
    dinv = dinv_ref[...]
    h = jnp.maximum((s0_ref[...] + s1_ref[...] + g_ref[...]) * dinv
                    + b_ref[...], 0.0)
    out_ref[...] = jnp.dot(h, w_ref[...],
                           preferred_element_type=jnp.float32) * dinv


_f32 = jnp.float32
_tc_first = pl.pallas_call(
    _tc_first_body,
    out_shape=[jax.ShapeDtypeStruct((PK, 128), _f32),
               jax.ShapeDtypeStruct((PK, 128), _f32)])
_tc_mid = pl.pallas_call(
    _tc_mid_body, out_shape=jax.ShapeDtypeStruct((PK, 128), _f32))
def _blockdiag(w):
    k = w.shape[0]
    return jnp.einsum("pq,kj->pkqj", jnp.eye(8, dtype=w.dtype),
                      w).reshape(8 * k, 128)


def kernel(x, edge_index, W0, b0, W1, b1, W2, b2, W3, b3, W4, b4, W5, b5,
           W6, b6, W7, b7):
    Ws = [W0, W1, W2, W3, W4, W5, W6, W7]
    bs = [b0, b1, b2, b3, b4, b5, b6, b7]

    src = edge_index[0]
    dst = edge_index[1]
    pad = EPAD - E
    srcs = jnp.concatenate(
        [src, jnp.full((pad,), DUMMY, jnp.int32)]).reshape(NTILES, NG, CG * CH)
    dsts = jnp.concatenate(
        [dst, jnp.full((pad,), DUMMY, jnp.int32)]).reshape(NTILES, NG, CG * CH)
    zeros = jnp.zeros((NP, D), _f32)
    ones = jnp.ones((NP, D), _f32)
    x_pp = jnp.pad(x, ((0, NP - N), (0, 0))).reshape(PK, 1024)

    w0big = _blockdiag(W0)
    wbigs = [_blockdiag(w) for w in Ws[1:]]
    btiles = [jnp.tile(b, 8).reshape(1, 128) for b in bs]

    aggones = _propagate(ones, srcs, dsts, zeros).reshape(2, PK, 128)

    dinv_p, g_p = _tc_first(aggones[0], aggones[1], x_pp, w0big)

    for i in range(8):
        g_flat = g_p.reshape(NP, D)
        sp_raw = _propagate(g_flat, srcs, dsts, zeros)
        if i < 7:
            sp = sp_raw.reshape(2, PK, 128)
            g_p = _tc_mid(sp[0], sp[1], g_p, dinv_p, btiles[i], wbigs[i])
        else:
            out = _sc_final(sp_raw, g_flat, dinv_p.reshape(NP, D), bs[i])

    return out
